# baseline (speedup 1.0000x reference)
import functools
import math

import jax
import jax.numpy as jnp
import numpy as np
from jax import lax
from jax.experimental import pallas as pl
from jax.experimental.pallas import tpu as pltpu
from jax.experimental.pallas import tpu_sc as plsc

NUM_EMBEDDINGS = 1000000
D = 64
SCALE = math.sqrt(D)

NW = 32
CHUNK = 100
N_CHUNKS = 64
PER_W = CHUNK * N_CHUNKS
B = NW * PER_W
NBUF = 8
N_ROUNDS = N_CHUNKS // NBUF


def _pe_block() -> np.ndarray:
    pos = np.arange(0, 50).reshape(-1, 1).astype(np.float32)
    even = np.arange(0, D, 2).astype(np.float32)
    power = -even * math.log(10000.0) / D
    pe = np.zeros((50, D), dtype=np.float32)
    pe[:, 0::2] = np.sin(pos * np.exp(power))
    pe[:, 1::2] = np.cos(pos * np.exp(power))
    return np.tile(pe, (CHUNK // 50, 1))


_PE = _pe_block()

_mesh = plsc.VectorSubcoreMesh(core_axis_name="c", subcore_axis_name="s")


DP = 128


@functools.partial(
    pl.kernel,
    out_type=jax.ShapeDtypeStruct((B // CHUNK, CHUNK, D), jnp.float32),
    mesh=_mesh,
    compiler_params=pltpu.CompilerParams(use_tc_tiling_on_sc=False),
    scratch_types=[
        pltpu.VMEM((N_CHUNKS, CHUNK), jnp.int32),
        pltpu.VMEM((CHUNK, D), jnp.float32),
        pltpu.VMEM((NBUF, CHUNK, DP), jnp.float32),
    ]
    + [pltpu.SemaphoreType.DMA] * (2 * NBUF),
)
def _emb_sc(idx_hbm, lut_hbm, pe_hbm, out_hbm, idx_v, pe_v, rows_v, *sems):
    gsems = sems[:NBUF]
    wsems = sems[NBUF:]
    wid = lax.axis_index("s") * 2 + lax.axis_index("c")
    pltpu.sync_copy(idx_hbm.at[wid], idx_v)
    pltpu.sync_copy(pe_hbm, pe_v)

    for b in range(NBUF):
        pltpu.async_copy(lut_hbm.at[idx_v.at[b]], rows_v.at[b], gsems[b])

    def round_body(r, carry):
        for b in range(NBUF):
            j = r * NBUF + b
            pltpu.make_async_copy(
                lut_hbm.at[idx_v.at[j]], rows_v.at[b], gsems[b]
            ).wait()

            def row_body(rr, c2):
                for cc in range(D // 16):
                    sl = pl.ds(cc * 16, 16)
                    rows_v[b, rr, sl] = rows_v[b, rr, sl] * SCALE + pe_v[rr, sl]
                return c2

            lax.fori_loop(0, CHUNK, row_body, 0)

            src = rows_v.at[b, :, pl.ds(0, D)]
            out_slot = out_hbm.at[wid * N_CHUNKS + j]
            pltpu.async_copy(src, out_slot, wsems[b])

            @pl.when(r < N_ROUNDS - 1)
            def _():
                pltpu.make_async_copy(src, out_slot, wsems[b]).wait()
                pltpu.async_copy(
                    lut_hbm.at[idx_v.at[j + NBUF]], rows_v.at[b], gsems[b]
                )

        return carry

    lax.fori_loop(0, N_ROUNDS, round_body, 0)

    for b in range(NBUF):
        j = (N_ROUNDS - 1) * NBUF + b
        pltpu.make_async_copy(
            rows_v.at[b, :, pl.ds(0, D)], out_hbm.at[wid * N_CHUNKS + j], wsems[b]
        ).wait()


def kernel(x, lut):
    n_seq, seq_len = x.shape
    idx = x.astype(jnp.int32).reshape(NW, N_CHUNKS, CHUNK)
    lut_p = jnp.pad(lut, ((0, 0), (0, DP - D)))
    out = _emb_sc(idx, lut_p, jnp.asarray(_PE))
    return out.reshape(n_seq, seq_len, D)


# speedup vs baseline: 1.2264x; 1.2264x over previous
import functools
import math

import jax
import jax.numpy as jnp
import numpy as np
from jax import lax
from jax.experimental import pallas as pl
from jax.experimental.pallas import tpu as pltpu
from jax.experimental.pallas import tpu_sc as plsc

NUM_EMBEDDINGS = 1000000
D = 64
SCALE = math.sqrt(D)
SEQ = 50
NSEQ = 4096

NW = 32
IBLK = NSEQ // NW
NBUF = 5
N_ROUNDS = SEQ // NBUF

TBLK = 4096
FMT_GRID = (NUM_EMBEDDINGS + TBLK - 1) // TBLK


def _pe_block() -> np.ndarray:
    pos = np.arange(0, SEQ).reshape(-1, 1).astype(np.float32)
    even = np.arange(0, D, 2).astype(np.float32)
    power = -even * math.log(10000.0) / D
    pe = np.zeros((SEQ, D), dtype=np.float32)
    pe[:, 0::2] = np.sin(pos * np.exp(power))
    pe[:, 1::2] = np.cos(pos * np.exp(power))
    return pe


_PE = _pe_block()


def _fmt_body(in_ref, out_ref):
    out_ref[:, 0:D] = in_ref[...].T


_fmt = pl.pallas_call(
    _fmt_body,
    grid=(FMT_GRID,),
    in_specs=[pl.BlockSpec((D, TBLK), lambda g: (0, g))],
    out_specs=pl.BlockSpec((TBLK, 2 * D), lambda g: (g, 0)),
    out_shape=jax.ShapeDtypeStruct((NUM_EMBEDDINGS, 2 * D), jnp.float32),
)

_mesh = plsc.VectorSubcoreMesh(core_axis_name="c", subcore_axis_name="s")


@functools.partial(
    pl.kernel,
    out_type=jax.ShapeDtypeStruct((SEQ, D // 8, NW, 8, IBLK), jnp.float32),
    mesh=_mesh,
    compiler_params=pltpu.CompilerParams(
        use_tc_tiling_on_sc=False, needs_layout_passes=False
    ),
    scratch_types=[
        pltpu.VMEM((SEQ, IBLK), jnp.int32),
        pltpu.VMEM((SEQ, D), jnp.float32),
        pltpu.VMEM((NBUF, IBLK, D), jnp.float32),
        pltpu.VMEM((NBUF, D // 8, 8, IBLK), jnp.float32),
    ]
    + [pltpu.SemaphoreType.DMA] * (2 * NBUF),
)
def _emb_sc(idx_hbm, lut_hbm, pe_hbm, out_hbm, idx_v, pe_v, rows_v, tout_v, *sems):
    gsems = sems[:NBUF]
    wsems = sems[NBUF:]
    wid = lax.axis_index("s") * 2 + lax.axis_index("c")
    pltpu.sync_copy(idx_hbm.at[wid], idx_v)
    pltpu.sync_copy(pe_hbm, pe_v)

    def start_gather(j, b):
        pltpu.async_copy(lut_hbm.at[idx_v.at[j]], rows_v.at[b], gsems[b])

    def wait_gather(j, b):
        pltpu.make_async_copy(
            lut_hbm.at[idx_v.at[j]], rows_v.at[b], gsems[b]
        ).wait()

    def start_writes(j, b):
        for co in range(D // 8):
            pltpu.async_copy(
                tout_v.at[b, co], out_hbm.at[j, co, wid], wsems[b]
            )

    def wait_writes(j, b):
        for co in range(D // 8):
            pltpu.make_async_copy(
                tout_v.at[b, co], out_hbm.at[j, co, wid], wsems[b]
            ).wait()

    for b in range(NBUF):
        start_gather(b, b)

    def round_body(r, carry):
        for b in range(NBUF):
            j = r * NBUF + b
            wait_gather(j, b)

            def row_body(i, pevs):
                for cc in range(D // 16):
                    sl = pl.ds(cc * 16, 16)
                    vals = rows_v[b, i, sl] * SCALE + pevs[cc]
                    cidx = jax.lax.iota(jnp.int32, 16) + cc * 16
                    plsc.store_scatter(
                        tout_v.at[b],
                        [
                            jax.lax.shift_right_logical(cidx, 3),
                            jax.lax.bitwise_and(cidx, 7),
                            jnp.full((16,), i, jnp.int32),
                        ],
                        vals,
                    )
                return pevs

            pevs = tuple(
                pe_v[j, pl.ds(cc * 16, 16)] for cc in range(D // 16)
            )
            lax.fori_loop(0, IBLK, row_body, pevs)

            start_writes(j, b)

            @pl.when(r < N_ROUNDS - 1)
            def _():
                wait_writes(j, b)
                start_gather(j + NBUF, b)

        return carry

    lax.fori_loop(0, N_ROUNDS, round_body, 0)

    for b in range(NBUF):
        wait_writes((N_ROUNDS - 1) * NBUF + b, b)


def kernel(x, lut):
    idx = (x.astype(jnp.int32) * 2).T.reshape(SEQ, NW, IBLK).transpose(1, 0, 2)
    lut_f = _fmt(lut.T).reshape(2 * NUM_EMBEDDINGS, D)
    out5 = _emb_sc(idx, lut_f, jnp.asarray(_PE))
    return out5.transpose(2, 4, 0, 1, 3).reshape(NSEQ, SEQ, D)


# speedup vs baseline: 1.2780x; 1.0421x over previous
import functools
import math

import jax
import jax.numpy as jnp
import numpy as np
from jax import lax
from jax.experimental import pallas as pl
from jax.experimental.pallas import tpu as pltpu
from jax.experimental.pallas import tpu_sc as plsc

NUM_EMBEDDINGS = 1000000
D = 64
SCALE = math.sqrt(D)
SEQ = 50
NSEQ = 4096

NW = 32
IBLK = NSEQ // NW
NBUF = 5
N_ROUNDS = SEQ // NBUF

TBLK = 4096
FMT_GRID = (NUM_EMBEDDINGS + TBLK - 1) // TBLK

TBLK2 = 512


def _pe_block() -> np.ndarray:
    pos = np.arange(0, SEQ).reshape(-1, 1).astype(np.float32)
    even = np.arange(0, D, 2).astype(np.float32)
    power = -even * math.log(10000.0) / D
    pe = np.zeros((SEQ, D), dtype=np.float32)
    pe[:, 0::2] = np.sin(pos * np.exp(power))
    pe[:, 1::2] = np.cos(pos * np.exp(power))
    return pe


_PE = _pe_block()


def _fmt_body(in_ref, out_ref):
    out_ref[:, 0:D] = in_ref[...].T


_fmt = pl.pallas_call(
    _fmt_body,
    grid=(FMT_GRID,),
    in_specs=[pl.BlockSpec((D, TBLK), lambda g: (0, g))],
    out_specs=pl.BlockSpec((TBLK, 2 * D), lambda g: (g, 0)),
    out_shape=jax.ShapeDtypeStruct((NUM_EMBEDDINGS, 2 * D), jnp.float32),
)


def _trans_body(in_ref, out_ref):
    blk = in_ref[0]
    out_ref[0] = jnp.concatenate([blk[:, 0:D].T, blk[:, D : 2 * D].T], axis=1)


_trans = pl.pallas_call(
    _trans_body,
    grid=(SEQ, NSEQ // TBLK2),
    in_specs=[pl.BlockSpec((1, TBLK2 // 2, 128), lambda j, t: (j, t, 0))],
    out_specs=pl.BlockSpec((1, D, TBLK2), lambda j, t: (j, 0, t)),
    out_shape=jax.ShapeDtypeStruct((SEQ, D, NSEQ), jnp.float32),
)

_mesh = plsc.VectorSubcoreMesh(core_axis_name="c", subcore_axis_name="s")


@functools.partial(
    pl.kernel,
    out_type=jax.ShapeDtypeStruct((SEQ, NW, IBLK, D), jnp.float32),
    mesh=_mesh,
    compiler_params=pltpu.CompilerParams(use_tc_tiling_on_sc=False),
    scratch_types=[
        pltpu.VMEM((SEQ, IBLK), jnp.int32),
        pltpu.VMEM((SEQ, D), jnp.float32),
        pltpu.VMEM((NBUF, IBLK, D), jnp.float32),
    ]
    + [pltpu.SemaphoreType.DMA] * (2 * NBUF),
)
def _emb_sc(idx_hbm, lut_hbm, pe_hbm, out_hbm, idx_v, pe_v, rows_v, *sems):
    gsems = sems[:NBUF]
    wsems = sems[NBUF:]
    wid = lax.axis_index("s") * 2 + lax.axis_index("c")
    pltpu.sync_copy(idx_hbm.at[wid], idx_v)
    pltpu.sync_copy(pe_hbm, pe_v)

    for b in range(NBUF):
        pltpu.async_copy(lut_hbm.at[idx_v.at[b]], rows_v.at[b], gsems[b])

    def round_body(r, carry):
        for b in range(NBUF):
            j = r * NBUF + b
            pltpu.make_async_copy(
                lut_hbm.at[idx_v.at[j]], rows_v.at[b], gsems[b]
            ).wait()

            def row_body(i, pevs):
                for cc in range(D // 16):
                    sl = pl.ds(cc * 16, 16)
                    rows_v[b, i, sl] = rows_v[b, i, sl] * SCALE + pevs[cc]
                return pevs

            pevs = tuple(pe_v[j, pl.ds(cc * 16, 16)] for cc in range(D // 16))
            lax.fori_loop(0, IBLK, row_body, pevs)

            out_slot = out_hbm.at[j, wid]
            pltpu.async_copy(rows_v.at[b], out_slot, wsems[b])

            @pl.when(r < N_ROUNDS - 1)
            def _():
                pltpu.make_async_copy(rows_v.at[b], out_slot, wsems[b]).wait()
                pltpu.async_copy(
                    lut_hbm.at[idx_v.at[j + NBUF]], rows_v.at[b], gsems[b]
                )

        return carry

    lax.fori_loop(0, N_ROUNDS, round_body, 0)

    for b in range(NBUF):
        j = (N_ROUNDS - 1) * NBUF + b
        pltpu.make_async_copy(
            rows_v.at[b], out_hbm.at[j, wid], wsems[b]
        ).wait()


def kernel(x, lut):
    idx = (x.astype(jnp.int32) * 2).T
    idx = idx.reshape(SEQ, NSEQ // TBLK2, 2, TBLK2 // 2).transpose(0, 1, 3, 2)
    idx = idx.reshape(SEQ, NW, IBLK).transpose(1, 0, 2)
    lut_f = _fmt(lut.T).reshape(2 * NUM_EMBEDDINGS, D)
    outj = _emb_sc(idx, lut_f, jnp.asarray(_PE))
    t3 = _trans(outj.reshape(SEQ, NSEQ * D // 128, 128))
    return t3.transpose(2, 0, 1)
